# baseline (speedup 1.0000x reference)
import numpy as np
import jax
import jax.numpy as jnp
from jax.experimental import pallas as pl
from jax.experimental.pallas import tpu as pltpu
from jax.experimental.pallas import tpu_sc as plsc

_B = 16384
_F = 26
_D = 64
_NF = _F + 1
_NPAIR = (_NF * (_NF - 1)) // 2

_GATHER_WINDOW = 128


def _sc_gather(table, flat_idx):
    n = flat_idx.shape[0]
    d = table.shape[1]
    idx2 = flat_idx.reshape(1, n)

    @jax.jit
    def run(table, idx2):
        @pl.kernel(
            out_type=jax.ShapeDtypeStruct((n, d), table.dtype),
            mesh=plsc.VectorSubcoreMesh(core_axis_name="core",
                                        subcore_axis_name="subcore"),
        )
        def gather_kernel(x_hbm, i_hbm, o_hbm):
            def body(i_vmem, o_vmem):
                pltpu.sync_copy(x_hbm.at[i_vmem.at[0]], o_vmem)

            pltpu.emit_pipeline(
                body,
                grid=(n // _GATHER_WINDOW,),
                in_specs=[pl.BlockSpec((1, _GATHER_WINDOW),
                                       index_map=lambda i: (0, i))],
                out_specs=[pl.BlockSpec((_GATHER_WINDOW, d),
                                        index_map=lambda i: (i, 0))],
                core_axis_name=("core", "subcore"),
                dimension_semantics=(pltpu.PARALLEL,),
            )(i_hbm, o_hbm)

        return gather_kernel(table, idx2)

    return run(table, idx2)


def _dense_mlp_kernel(x_ref, w1_ref, b1_ref, w2_ref, b2_ref, w3_ref, b3_ref,
                      o_ref):
    x = x_ref[...]
    h = jnp.maximum(x @ w1_ref[...] + b1_ref[...], 0.0)
    h = jnp.maximum(h @ w2_ref[...] + b2_ref[...], 0.0)
    h = jnp.maximum(h @ w3_ref[...] + b3_ref[...], 0.0)
    o_ref[...] = h


def _dense_mlp(x_pad, w1p, b1, w2, b2, w3, b3):
    bb = 2048
    full = lambda a: pl.BlockSpec(a.shape, lambda i: (0,) * a.ndim)
    return pl.pallas_call(
        _dense_mlp_kernel,
        grid=(_B // bb,),
        in_specs=[pl.BlockSpec((bb, x_pad.shape[1]), lambda i: (i, 0)),
                  full(w1p), full(b1), full(w2), full(b2), full(w3), full(b3)],
        out_specs=pl.BlockSpec((bb, _D), lambda i: (i, 0)),
        out_shape=jax.ShapeDtypeStruct((_B, _D), jnp.float32),
    )(x_pad, w1p, b1, w2, b2, w3, b3)


def _main_kernel(c_ref, w1_ref, b1_ref, w2_ref, b2_ref, w3_ref, b3_ref,
                 w4_ref, b4_ref, w5_ref, b5_ref, o_ref):
    c = c_ref[...]
    d = c[:, 0, :]
    inter = jax.lax.dot_general(
        c, c, dimension_numbers=(((2,), (2,)), ((0,), (0,))),
        preferred_element_type=jnp.float32)
    parts = [d]
    for i in range(1, _NF):
        parts.append(inter[:, i, :i])
    bb = c.shape[0]
    parts.append(jnp.zeros((bb, 1), jnp.float32))
    x = jnp.concatenate(parts, axis=1)
    x = jnp.maximum(x @ w1_ref[...] + b1_ref[...], 0.0)
    x = jnp.maximum(x @ w2_ref[...] + b2_ref[...], 0.0)
    x = jnp.maximum(x @ w3_ref[...] + b3_ref[...], 0.0)
    x = jnp.maximum(x @ w4_ref[...] + b4_ref[...], 0.0)
    o_ref[...] = x @ w5_ref[...] + b5_ref[...]


def _main(combined, w1p, b1, w2, b2, w3, b3, w4, b4, w5, b5):
    bb = 512
    full = lambda a: pl.BlockSpec(a.shape, lambda i: (0,) * a.ndim)
    return pl.pallas_call(
        _main_kernel,
        grid=(_B // bb,),
        in_specs=[pl.BlockSpec((bb, _NF, _D), lambda i: (i, 0, 0)),
                  full(w1p), full(b1), full(w2), full(b2), full(w3), full(b3),
                  full(w4), full(b4), full(w5), full(b5)],
        out_specs=pl.BlockSpec((bb, 1), lambda i: (i, 0)),
        out_shape=jax.ShapeDtypeStruct((_B, 1), jnp.float32),
    )(combined, w1p, b1, w2, b2, w3, b3, w4, b4, w5, b5)


def kernel(dense_features, sparse_indices, offsets, W_embed, dense_params,
           over_params):
    flat_idx = (sparse_indices + offsets[None, :]).reshape(-1).astype(jnp.int32)

    (w1d, b1d), (w2d, b2d), (w3d, b3d) = dense_params
    x_pad = jnp.pad(dense_features, ((0, 0), (0, 16 - dense_features.shape[1])))
    w1d_pad = jnp.pad(w1d, ((0, 16 - w1d.shape[0]), (0, 0)))

    (w1o, b1o), (w2o, b2o), (w3o, b3o), (w4o, b4o), (w5o, b5o) = over_params
    w1o_pad = jnp.pad(w1o, ((0, 416 - w1o.shape[0]), (0, 0)))

    r2 = lambda b: b.reshape(1, -1)

    emb = jnp.take(W_embed, flat_idx, axis=0)

    d = _dense_mlp(x_pad, w1d_pad, r2(b1d), w2d, r2(b2d), w3d, r2(b3d))

    combined = jnp.concatenate([d[:, None, :], emb.reshape(_B, _F, _D)], axis=1)
    logits = _main(combined, w1o_pad, r2(b1o), w2o, r2(b2o), w3o, r2(b3o),
                   w4o, r2(b4o), w5o, r2(b5o))
    return logits
